# Initial kernel scaffold; baseline (speedup 1.0000x reference)
import jax
import jax.numpy as jnp
from jax.experimental import pallas as pl


def kernel(x, edge_index, W_enc, b_enc, W_msg, W_self, b_out):
    raise NotImplementedError("write your pallas kernel here")


# speedup vs baseline: 5.6916x; 5.6916x over previous
import functools

import jax
import jax.numpy as jnp
from jax import lax
from jax.experimental import pallas as pl
from jax.experimental.pallas import tpu as pltpu
from jax.experimental.pallas import tpu_sc as plsc

N = 10000
E = 320000
D = 128
L = 128
DA = 144

NC = 2
NS = 16
NW = NC * NS
EPW = E // NW
C = 80
G = EPW // C
RPT = N // NS
RPTC = 125
KS = RPT // RPTC


def _enc_body(x_ref, w_ref, b_ref, out_ref):
    h = jnp.dot(x_ref[...], w_ref[...], preferred_element_type=jnp.float32)
    h = h + b_ref[...]
    h = jnp.where(h > 0, h, jnp.exp(h) - 1.0)
    out_ref[:, :D] = h
    lane = lax.broadcasted_iota(jnp.int32, (out_ref.shape[0], DA - D), 1)
    out_ref[:, D:] = jnp.where(lane == 0, 1.0, 0.0)


def _encode(x, w_enc, b_enc):
    R = 2000
    return pl.pallas_call(
        _enc_body,
        grid=(N // R,),
        in_specs=[
            pl.BlockSpec((R, D), lambda i: (i, 0)),
            pl.BlockSpec((D, L), lambda i: (0, 0)),
            pl.BlockSpec((1, L), lambda i: (0, 0)),
        ],
        out_specs=pl.BlockSpec((R, DA), lambda i: (i, 0)),
        out_shape=jax.ShapeDtypeStruct((N, DA), jnp.float32),
    )(x, w_enc, b_enc.reshape(1, L))


def _sc_body(h_ref, src_ref, dst_ref, out_ref, acc, stg, rows, isrc, idst, sem):
    cid = lax.axis_index("c")
    sid = lax.axis_index("s")
    wid = cid * NS + sid

    zero = jnp.zeros((16,), jnp.float32)

    def zrow(r, carry):
        for j in range(DA // 16):
            stg[r, pl.ds(j * 16, 16)] = zero
        return carry

    lax.fori_loop(0, RPTC, zrow, 0)
    row0 = sid * RPT

    def zcp(k, carry):
        pltpu.sync_copy(stg, acc.at[pl.ds(row0 + k * RPTC, RPTC)])
        return carry

    lax.fori_loop(0, KS, zcp, 0)
    plsc.subcore_barrier()

    def chunk(g, carry):
        off = pl.multiple_of(wid * EPW + g * C, 8)
        pltpu.sync_copy(src_ref.at[pl.ds(off, C)], isrc)
        pltpu.sync_copy(dst_ref.at[pl.ds(off, C)], idst)
        pltpu.async_copy(h_ref.at[isrc], rows, sem).wait()
        pltpu.sync_copy(rows, acc.at[idst], add=True)
        return carry

    lax.fori_loop(0, G, chunk, 0)
    plsc.subcore_barrier()

    def ocp(k, carry):
        r0 = row0 + k * RPTC
        pltpu.sync_copy(acc.at[pl.ds(r0, RPTC)], stg)
        pltpu.sync_copy(stg, out_ref.at[cid, pl.ds(r0, RPTC)])
        return carry

    lax.fori_loop(0, KS, ocp, 0)


def _scatter(h_aug, src, dst):
    mesh = plsc.VectorSubcoreMesh(core_axis_name="c", subcore_axis_name="s")
    return pl.kernel(
        _sc_body,
        out_type=jax.ShapeDtypeStruct((NC, N, DA), jnp.float32),
        mesh=mesh,
        compiler_params=pltpu.CompilerParams(use_tc_tiling_on_sc=False),
        scratch_types=[
            pltpu.VMEM_SHARED((N, DA), jnp.float32),
            pltpu.VMEM((RPTC, DA), jnp.float32),
            pltpu.VMEM((C, DA), jnp.float32),
            pltpu.VMEM((C,), jnp.int32),
            pltpu.VMEM((C,), jnp.int32),
            pltpu.SemaphoreType.DMA,
        ],
    )(h_aug, src, dst)


def _comb_body(p_ref, h_ref, wm_ref, ws_ref, b_ref, out_ref):
    s = p_ref[0] + p_ref[1]
    pre = s[:, :D]
    deg = jnp.maximum(s[:, D:D + 1], 1.0)
    agg = jnp.dot(pre, wm_ref[...], preferred_element_type=jnp.float32) / deg
    o = agg + jnp.dot(h_ref[:, :D], ws_ref[...],
                      preferred_element_type=jnp.float32) + b_ref[...]
    out_ref[...] = jnp.where(o > 0, o, jnp.exp(o) - 1.0)


def _combine(partials, h_aug, w_msg, w_self, b_out):
    R = 2000
    return pl.pallas_call(
        _comb_body,
        grid=(N // R,),
        in_specs=[
            pl.BlockSpec((NC, R, DA), lambda i: (0, i, 0)),
            pl.BlockSpec((R, DA), lambda i: (i, 0)),
            pl.BlockSpec((L, L), lambda i: (0, 0)),
            pl.BlockSpec((L, L), lambda i: (0, 0)),
            pl.BlockSpec((1, L), lambda i: (0, 0)),
        ],
        out_specs=pl.BlockSpec((R, L), lambda i: (i, 0)),
        out_shape=jax.ShapeDtypeStruct((N, L), jnp.float32),
    )(partials, h_aug, w_msg, w_self, b_out.reshape(1, L))


def kernel(x, edge_index, W_enc, b_enc, W_msg, W_self, b_out):
    src = edge_index[0].astype(jnp.int32)
    dst = edge_index[1].astype(jnp.int32)
    h_aug = _encode(x, W_enc, b_enc)
    partials = _scatter(h_aug, src, dst)
    return _combine(partials, h_aug, W_msg, W_self, b_out)


# speedup vs baseline: 8.3490x; 1.4669x over previous
import functools

import jax
import jax.numpy as jnp
from jax import lax
from jax.experimental import pallas as pl
from jax.experimental.pallas import tpu as pltpu
from jax.experimental.pallas import tpu_sc as plsc

N = 10000
E = 320000
D = 128
L = 128
DA = 144

NC = 2
NS = 16
NW = NC * NS
EPW = E // NW
C = 80
G = EPW // C
RPT = N // NS
RPTC = 25
KS = RPT // RPTC


def _enc_body(x_ref, w_ref, b_ref, out_ref):
    h = jnp.dot(x_ref[...], w_ref[...], preferred_element_type=jnp.float32)
    h = h + b_ref[...]
    h = jnp.where(h > 0, h, jnp.exp(h) - 1.0)
    out_ref[:, :D] = h
    lane = lax.broadcasted_iota(jnp.int32, (out_ref.shape[0], DA - D), 1)
    out_ref[:, D:] = jnp.where(lane == 0, 1.0, 0.0)


def _encode(x, w_enc, b_enc):
    R = 2000
    return pl.pallas_call(
        _enc_body,
        grid=(N // R,),
        in_specs=[
            pl.BlockSpec((R, D), lambda i: (i, 0)),
            pl.BlockSpec((D, L), lambda i: (0, 0)),
            pl.BlockSpec((1, L), lambda i: (0, 0)),
        ],
        out_specs=pl.BlockSpec((R, DA), lambda i: (i, 0)),
        out_shape=jax.ShapeDtypeStruct((N, DA), jnp.float32),
    )(x, w_enc, b_enc.reshape(1, L))


def _sc_body(h_ref, src_ref, dst_ref, out_ref, acc, stg,
             rows0, rows1, isrc0, isrc1, idst0, idst1, sem0, sem1):
    cid = lax.axis_index("c")
    sid = lax.axis_index("s")
    wid = cid * NS + sid

    zero = jnp.zeros((16,), jnp.float32)

    def zrow(r, carry):
        for j in range(DA // 16):
            stg[r, pl.ds(j * 16, 16)] = zero
        return carry

    lax.fori_loop(0, RPTC, zrow, 0)
    row0 = sid * RPT

    def zcp(k, carry):
        pltpu.sync_copy(stg, acc.at[pl.ds(row0 + k * RPTC, RPTC)])
        return carry

    lax.fori_loop(0, KS, zcp, 0)
    plsc.subcore_barrier()

    base = wid * EPW

    def launch(g, isrc, idst, rows, sem):
        off = pl.multiple_of(base + g * C, 8)
        pltpu.sync_copy(src_ref.at[pl.ds(off, C)], isrc)
        pltpu.sync_copy(dst_ref.at[pl.ds(off, C)], idst)
        pltpu.async_copy(h_ref.at[isrc], rows, sem)

    def drain(isrc, idst, rows, sem):
        pltpu.make_async_copy(h_ref.at[isrc], rows, sem).wait()
        pltpu.sync_copy(rows, acc.at[idst], add=True)

    launch(0, isrc0, idst0, rows0, sem0)

    def pair(i, carry):
        g = i * 2
        launch(g + 1, isrc1, idst1, rows1, sem1)
        drain(isrc0, idst0, rows0, sem0)
        launch(g + 2, isrc0, idst0, rows0, sem0)
        drain(isrc1, idst1, rows1, sem1)
        return carry

    lax.fori_loop(0, (G - 1) // 2, pair, 0)
    drain(isrc0, idst0, rows0, sem0)
    plsc.subcore_barrier()

    def ocp(k, carry):
        r0 = row0 + k * RPTC
        pltpu.sync_copy(acc.at[pl.ds(r0, RPTC)], stg)
        pltpu.sync_copy(stg, out_ref.at[cid, pl.ds(r0, RPTC)])
        return carry

    lax.fori_loop(0, KS, ocp, 0)


def _scatter(h_aug, src, dst):
    mesh = plsc.VectorSubcoreMesh(core_axis_name="c", subcore_axis_name="s")
    return pl.kernel(
        _sc_body,
        out_type=jax.ShapeDtypeStruct((NC, N, DA), jnp.float32),
        mesh=mesh,
        compiler_params=pltpu.CompilerParams(use_tc_tiling_on_sc=False),
        scratch_types=[
            pltpu.VMEM_SHARED((N, DA), jnp.float32),
            pltpu.VMEM((RPTC, DA), jnp.float32),
            pltpu.VMEM((C, DA), jnp.float32),
            pltpu.VMEM((C, DA), jnp.float32),
            pltpu.VMEM((C,), jnp.int32),
            pltpu.VMEM((C,), jnp.int32),
            pltpu.VMEM((C,), jnp.int32),
            pltpu.VMEM((C,), jnp.int32),
            pltpu.SemaphoreType.DMA,
            pltpu.SemaphoreType.DMA,
        ],
    )(h_aug, src, dst)


def _comb_body(p_ref, h_ref, wm_ref, ws_ref, b_ref, out_ref):
    s = p_ref[0] + p_ref[1]
    pre = s[:, :D]
    deg = jnp.maximum(s[:, D:D + 1], 1.0)
    agg = jnp.dot(pre, wm_ref[...], preferred_element_type=jnp.float32) / deg
    o = agg + jnp.dot(h_ref[:, :D], ws_ref[...],
                      preferred_element_type=jnp.float32) + b_ref[...]
    out_ref[...] = jnp.where(o > 0, o, jnp.exp(o) - 1.0)


def _combine(partials, h_aug, w_msg, w_self, b_out):
    R = 2000
    return pl.pallas_call(
        _comb_body,
        grid=(N // R,),
        in_specs=[
            pl.BlockSpec((NC, R, DA), lambda i: (0, i, 0)),
            pl.BlockSpec((R, DA), lambda i: (i, 0)),
            pl.BlockSpec((L, L), lambda i: (0, 0)),
            pl.BlockSpec((L, L), lambda i: (0, 0)),
            pl.BlockSpec((1, L), lambda i: (0, 0)),
        ],
        out_specs=pl.BlockSpec((R, L), lambda i: (i, 0)),
        out_shape=jax.ShapeDtypeStruct((N, L), jnp.float32),
    )(partials, h_aug, w_msg, w_self, b_out.reshape(1, L))


def kernel(x, edge_index, W_enc, b_enc, W_msg, W_self, b_out):
    src = edge_index[0].astype(jnp.int32)
    dst = edge_index[1].astype(jnp.int32)
    h_aug = _encode(x, W_enc, b_enc)
    partials = _scatter(h_aug, src, dst)
    return _combine(partials, h_aug, W_msg, W_self, b_out)


# speedup vs baseline: 10.9001x; 1.3056x over previous
import functools

import jax
import jax.numpy as jnp
from jax import lax
from jax.experimental import pallas as pl
from jax.experimental.pallas import tpu as pltpu
from jax.experimental.pallas import tpu_sc as plsc

N = 10000
E = 320000
D = 128
L = 128
DA = 144

NC = 2
NS = 16
NW = NC * NS
EPW = E // NW
C = 80
G = EPW // C
B = 25
NB = G // B
RPT = N // NS
RPTC = 25
KS = RPT // RPTC


def _enc_body(x_ref, w_ref, b_ref, out_ref):
    h = jnp.dot(x_ref[...], w_ref[...], preferred_element_type=jnp.float32)
    h = h + b_ref[...]
    h = jnp.where(h > 0, h, jnp.exp(h) - 1.0)
    out_ref[:, :D] = h
    lane = lax.broadcasted_iota(jnp.int32, (out_ref.shape[0], DA - D), 1)
    out_ref[:, D:] = jnp.where(lane == 0, 1.0, 0.0)


def _encode(x, w_enc, b_enc):
    R = 2000
    return pl.pallas_call(
        _enc_body,
        grid=(N // R,),
        in_specs=[
            pl.BlockSpec((R, D), lambda i: (i, 0)),
            pl.BlockSpec((D, L), lambda i: (0, 0)),
            pl.BlockSpec((1, L), lambda i: (0, 0)),
        ],
        out_specs=pl.BlockSpec((R, DA), lambda i: (i, 0)),
        out_shape=jax.ShapeDtypeStruct((N, DA), jnp.float32),
    )(x, w_enc, b_enc.reshape(1, L))


def _sc_body(h_ref, src_ref, dst_ref, out_ref, acc, stg,
             rows0, rows1, isA, idA, isB, idB, sem0, sem1, isem0, isem1):
    cid = lax.axis_index("c")
    sid = lax.axis_index("s")
    wid = cid * NS + sid

    zero = jnp.zeros((16,), jnp.float32)

    def zrow(r, carry):
        for j in range(DA // 16):
            stg[r, pl.ds(j * 16, 16)] = zero
        return carry

    lax.fori_loop(0, RPTC, zrow, 0)
    row0 = sid * RPT

    def zcp(k, carry):
        pltpu.sync_copy(stg, acc.at[pl.ds(row0 + k * RPTC, RPTC)])
        return carry

    lax.fori_loop(0, KS, zcp, 0)
    plsc.subcore_barrier()

    brow = wid * G

    def iload(b, sb, db, isem):
        r = brow + b * B
        pltpu.async_copy(src_ref.at[pl.ds(r, B)], sb, isem)
        pltpu.async_copy(dst_ref.at[pl.ds(r, B)], db, isem)

    def iwait(b, sb, db, isem):
        r = brow + b * B
        pltpu.make_async_copy(src_ref.at[pl.ds(r, B)], sb, isem).wait()
        pltpu.make_async_copy(dst_ref.at[pl.ds(r, B)], db, isem).wait()

    def fire(sb, j, rows, sem):
        pltpu.async_copy(h_ref.at[sb.at[j]], rows, sem)

    def drain(sb, db, j, rows, sem):
        pltpu.make_async_copy(h_ref.at[sb.at[j]], rows, sem).wait()
        pltpu.sync_copy(rows, acc.at[db.at[j]], add=True)

    rbufs = (rows0, rows1)
    sems = (sem0, sem1)
    ibufs = ((isA, idA, isem0), (isB, idB, isem1))

    iload(0, isA, idA, isem0)
    iwait(0, isA, idA, isem0)
    iload(1, isB, idB, isem1)
    fire(isA, 0, rows0, sem0)

    for b in range(NB):
        sb, db, _ = ibufs[b % 2]
        p = b % 2

        def pair(i, carry, sb=sb, db=db, p=p):
            j = i * 2
            fire(sb, j + 1, rbufs[1 - p], sems[1 - p])
            drain(sb, db, j, rbufs[p], sems[p])
            fire(sb, j + 2, rbufs[p], sems[p])
            drain(sb, db, j + 1, rbufs[1 - p], sems[1 - p])
            return carry

        lax.fori_loop(0, (B - 1) // 2, pair, 0)
        if b + 1 < NB:
            nsb, ndb, nisem = ibufs[(b + 1) % 2]
            iwait(b + 1, nsb, ndb, nisem)
            fire(nsb, 0, rbufs[1 - p], sems[1 - p])
            drain(sb, db, B - 1, rbufs[p], sems[p])
            if b + 2 < NB:
                iload(b + 2, sb, db, ibufs[b % 2][2])
        else:
            drain(sb, db, B - 1, rbufs[p], sems[p])
    plsc.subcore_barrier()

    def ocp(k, carry):
        r0 = row0 + k * RPTC
        pltpu.sync_copy(acc.at[pl.ds(r0, RPTC)], stg)
        pltpu.sync_copy(stg, out_ref.at[cid, pl.ds(r0, RPTC)])
        return carry

    lax.fori_loop(0, KS, ocp, 0)


def _scatter(h_aug, src, dst):
    mesh = plsc.VectorSubcoreMesh(core_axis_name="c", subcore_axis_name="s")
    return pl.kernel(
        _sc_body,
        out_type=jax.ShapeDtypeStruct((NC, N, DA), jnp.float32),
        mesh=mesh,
        compiler_params=pltpu.CompilerParams(use_tc_tiling_on_sc=False),
        scratch_types=[
            pltpu.VMEM_SHARED((N, DA), jnp.float32),
            pltpu.VMEM((RPTC, DA), jnp.float32),
            pltpu.VMEM((C, DA), jnp.float32),
            pltpu.VMEM((C, DA), jnp.float32),
            pltpu.VMEM((B, C), jnp.int32),
            pltpu.VMEM((B, C), jnp.int32),
            pltpu.VMEM((B, C), jnp.int32),
            pltpu.VMEM((B, C), jnp.int32),
            pltpu.SemaphoreType.DMA,
            pltpu.SemaphoreType.DMA,
            pltpu.SemaphoreType.DMA,
            pltpu.SemaphoreType.DMA,
        ],
    )(h_aug, src, dst)


def _comb_body(p_ref, h_ref, wm_ref, ws_ref, b_ref, out_ref):
    s = p_ref[0] + p_ref[1]
    pre = s[:, :D]
    deg = jnp.maximum(s[:, D:D + 1], 1.0)
    agg = jnp.dot(pre, wm_ref[...], preferred_element_type=jnp.float32) / deg
    o = agg + jnp.dot(h_ref[:, :D], ws_ref[...],
                      preferred_element_type=jnp.float32) + b_ref[...]
    out_ref[...] = jnp.where(o > 0, o, jnp.exp(o) - 1.0)


def _combine(partials, h_aug, w_msg, w_self, b_out):
    R = 2000
    return pl.pallas_call(
        _comb_body,
        grid=(N // R,),
        in_specs=[
            pl.BlockSpec((NC, R, DA), lambda i: (0, i, 0)),
            pl.BlockSpec((R, DA), lambda i: (i, 0)),
            pl.BlockSpec((L, L), lambda i: (0, 0)),
            pl.BlockSpec((L, L), lambda i: (0, 0)),
            pl.BlockSpec((1, L), lambda i: (0, 0)),
        ],
        out_specs=pl.BlockSpec((R, L), lambda i: (i, 0)),
        out_shape=jax.ShapeDtypeStruct((N, L), jnp.float32),
    )(partials, h_aug, w_msg, w_self, b_out.reshape(1, L))


def kernel(x, edge_index, W_enc, b_enc, W_msg, W_self, b_out):
    src = edge_index[0].astype(jnp.int32).reshape(E // C, C)
    dst = edge_index[1].astype(jnp.int32).reshape(E // C, C)
    h_aug = _encode(x, W_enc, b_enc)
    partials = _scatter(h_aug, src, dst)
    return _combine(partials, h_aug, W_msg, W_self, b_out)


# speedup vs baseline: 12.2425x; 1.1232x over previous
import functools

import jax
import jax.numpy as jnp
from jax import lax
from jax.experimental import pallas as pl
from jax.experimental.pallas import tpu as pltpu
from jax.experimental.pallas import tpu_sc as plsc

N = 10000
E = 320000
D = 128
L = 128
DG = 16

NC = 2
NS = 16
NW = NC * NS
EPW = E // NW
C = 80
G = EPW // C
B = 25
NB = G // B
RPT = N // NS
RPTC = 25
KS = RPT // RPTC


def _enc_body(x_ref, w_ref, b_ref, out_ref):
    h = jnp.dot(x_ref[...], w_ref[...], preferred_element_type=jnp.float32)
    h = h + b_ref[...]
    out_ref[...] = jnp.where(h > 0, h, jnp.exp(h) - 1.0)


def _encode(x, w_enc, b_enc):
    R = 2000
    return pl.pallas_call(
        _enc_body,
        grid=(N // R,),
        in_specs=[
            pl.BlockSpec((R, D), lambda i: (i, 0)),
            pl.BlockSpec((D, L), lambda i: (0, 0)),
            pl.BlockSpec((1, L), lambda i: (0, 0)),
        ],
        out_specs=pl.BlockSpec((R, D), lambda i: (i, 0)),
        out_shape=jax.ShapeDtypeStruct((N, D), jnp.float32),
    )(x, w_enc, b_enc.reshape(1, L))


def _sc_body(h_ref, src_ref, dst_ref, out_ref, dout_ref, acc, dacc, stg, dstg,
             ones, rows0, rows1, isA, idA, isB, idB, sem0, sem1, isem0, isem1):
    cid = lax.axis_index("c")
    sid = lax.axis_index("s")
    wid = cid * NS + sid

    lane = lax.iota(jnp.int32, 16)
    degpat = jnp.where(lane == 0, 1.0, 0.0).astype(jnp.float32)
    zero = jnp.zeros((16,), jnp.float32)

    def frow(r, carry):
        ones[r, pl.ds(0, DG)] = degpat
        return carry

    lax.fori_loop(0, C, frow, 0)

    def zrow(r, carry):
        for j in range(D // 16):
            stg[r, pl.ds(j * 16, 16)] = zero
        dstg[r, pl.ds(0, DG)] = zero
        return carry

    lax.fori_loop(0, RPTC, zrow, 0)
    row0 = sid * RPT

    def zcp(k, carry):
        pltpu.sync_copy(stg, acc.at[pl.ds(row0 + k * RPTC, RPTC)])
        pltpu.sync_copy(dstg, dacc.at[pl.ds(row0 + k * RPTC, RPTC)])
        return carry

    lax.fori_loop(0, KS, zcp, 0)
    plsc.subcore_barrier()

    brow = wid * G

    def iload(b, sb, db, isem):
        r = brow + b * B
        pltpu.async_copy(src_ref.at[pl.ds(r, B)], sb, isem)
        pltpu.async_copy(dst_ref.at[pl.ds(r, B)], db, isem)

    def iwait(b, sb, db, isem):
        r = brow + b * B
        pltpu.make_async_copy(src_ref.at[pl.ds(r, B)], sb, isem).wait()
        pltpu.make_async_copy(dst_ref.at[pl.ds(r, B)], db, isem).wait()

    def fire(sb, j, rows, sem):
        pltpu.async_copy(h_ref.at[sb.at[j]], rows, sem)

    def drain(sb, db, j, rows, sem):
        pltpu.make_async_copy(h_ref.at[sb.at[j]], rows, sem).wait()
        pltpu.sync_copy(rows, acc.at[db.at[j]], add=True)
        pltpu.sync_copy(ones, dacc.at[db.at[j]], add=True)

    rbufs = (rows0, rows1)
    sems = (sem0, sem1)
    ibufs = ((isA, idA, isem0), (isB, idB, isem1))

    iload(0, isA, idA, isem0)
    iwait(0, isA, idA, isem0)
    iload(1, isB, idB, isem1)
    fire(isA, 0, rows0, sem0)

    for b in range(NB):
        sb, db, _ = ibufs[b % 2]
        p = b % 2

        def pair(i, carry, sb=sb, db=db, p=p):
            j = i * 2
            fire(sb, j + 1, rbufs[1 - p], sems[1 - p])
            drain(sb, db, j, rbufs[p], sems[p])
            fire(sb, j + 2, rbufs[p], sems[p])
            drain(sb, db, j + 1, rbufs[1 - p], sems[1 - p])
            return carry

        lax.fori_loop(0, (B - 1) // 2, pair, 0)
        if b + 1 < NB:
            nsb, ndb, nisem = ibufs[(b + 1) % 2]
            iwait(b + 1, nsb, ndb, nisem)
            fire(nsb, 0, rbufs[1 - p], sems[1 - p])
            drain(sb, db, B - 1, rbufs[p], sems[p])
            if b + 2 < NB:
                iload(b + 2, sb, db, ibufs[b % 2][2])
        else:
            drain(sb, db, B - 1, rbufs[p], sems[p])
    plsc.subcore_barrier()

    def ocp(k, carry):
        r0 = row0 + k * RPTC
        pltpu.sync_copy(acc.at[pl.ds(r0, RPTC)], stg)
        pltpu.sync_copy(stg, out_ref.at[cid, pl.ds(r0, RPTC)])
        pltpu.sync_copy(dacc.at[pl.ds(r0, RPTC)], dstg)
        pltpu.sync_copy(dstg, dout_ref.at[cid, pl.ds(r0, RPTC)])
        return carry

    lax.fori_loop(0, KS, ocp, 0)


def _scatter(h_aug, src, dst):
    mesh = plsc.VectorSubcoreMesh(core_axis_name="c", subcore_axis_name="s")
    return pl.kernel(
        _sc_body,
        out_type=(
            jax.ShapeDtypeStruct((NC, N, D), jnp.float32),
            jax.ShapeDtypeStruct((NC, N, DG), jnp.float32),
        ),
        mesh=mesh,
        compiler_params=pltpu.CompilerParams(use_tc_tiling_on_sc=False),
        scratch_types=[
            pltpu.VMEM_SHARED((N, D), jnp.float32),
            pltpu.VMEM_SHARED((N, DG), jnp.float32),
            pltpu.VMEM((RPTC, D), jnp.float32),
            pltpu.VMEM((RPTC, DG), jnp.float32),
            pltpu.VMEM((C, DG), jnp.float32),
            pltpu.VMEM((C, D), jnp.float32),
            pltpu.VMEM((C, D), jnp.float32),
            pltpu.VMEM((B, C), jnp.int32),
            pltpu.VMEM((B, C), jnp.int32),
            pltpu.VMEM((B, C), jnp.int32),
            pltpu.VMEM((B, C), jnp.int32),
            pltpu.SemaphoreType.DMA,
            pltpu.SemaphoreType.DMA,
            pltpu.SemaphoreType.DMA,
            pltpu.SemaphoreType.DMA,
        ],
    )(h_aug, src, dst)


def _comb_body(p_ref, dp_ref, h_ref, wm_ref, ws_ref, b_ref, out_ref):
    pre = p_ref[0] + p_ref[1]
    deg = jnp.maximum(dp_ref[0, :, 0:1] + dp_ref[1, :, 0:1], 1.0)
    agg = jnp.dot(pre, wm_ref[...], preferred_element_type=jnp.float32) / deg
    o = agg + jnp.dot(h_ref[...], ws_ref[...],
                      preferred_element_type=jnp.float32) + b_ref[...]
    out_ref[...] = jnp.where(o > 0, o, jnp.exp(o) - 1.0)


def _combine(partials, dpartials, h, w_msg, w_self, b_out):
    R = 2000
    return pl.pallas_call(
        _comb_body,
        grid=(N // R,),
        in_specs=[
            pl.BlockSpec((NC, R, D), lambda i: (0, i, 0)),
            pl.BlockSpec((NC, R, DG), lambda i: (0, i, 0)),
            pl.BlockSpec((R, D), lambda i: (i, 0)),
            pl.BlockSpec((L, L), lambda i: (0, 0)),
            pl.BlockSpec((L, L), lambda i: (0, 0)),
            pl.BlockSpec((1, L), lambda i: (0, 0)),
        ],
        out_specs=pl.BlockSpec((R, L), lambda i: (i, 0)),
        out_shape=jax.ShapeDtypeStruct((N, L), jnp.float32),
    )(partials, dpartials, h, w_msg, w_self, b_out.reshape(1, L))


def kernel(x, edge_index, W_enc, b_enc, W_msg, W_self, b_out):
    src = edge_index[0].astype(jnp.int32).reshape(E // C, C)
    dst = edge_index[1].astype(jnp.int32).reshape(E // C, C)
    h = _encode(x, W_enc, b_enc)
    partials, dpartials = _scatter(h, src, dst)
    return _combine(partials, dpartials, h, W_msg, W_self, b_out)
